# Initial kernel scaffold; baseline (speedup 1.0000x reference)
import jax
import jax.numpy as jnp
from jax.experimental import pallas as pl


def kernel(x, W1s, b1s, W2s, b2s, W1r, b1r, W2r, b2r, Wr, br):
    raise NotImplementedError("write your pallas kernel here")


# speedup vs baseline: 2.1313x; 2.1313x over previous
import functools

import jax
import jax.numpy as jnp
from jax.experimental import pallas as pl
from jax.experimental.pallas import tpu as pltpu

DIM = 1024
INTER = 1024
NE = 15
TOPK = 7
SEQ = 2048
LANES = 128

_NEG = -1e30


def _gelu_exact(h):
    return 0.5 * h * (1.0 + jax.lax.erf(h * 0.7071067811865476))


def _k1_body(x_ref, xb_ref, w1s_ref, b1s_ref, w2s_ref, b2s_ref,
             wr_ref, brp_ref, y0_ref, gates_ref):
    h = jnp.dot(xb_ref[...], w1s_ref[...], preferred_element_type=jnp.float32)
    h = _gelu_exact(h + b1s_ref[...])
    sh = jnp.dot(h.astype(jnp.bfloat16), w2s_ref[...],
                 preferred_element_type=jnp.float32) + b2s_ref[...]
    y0_ref[...] = x_ref[...] + sh

    logits = jnp.dot(x_ref[...], wr_ref[...],
                     preferred_element_type=jnp.float32) + brp_ref[...]
    m = jnp.max(logits, axis=-1, keepdims=True)
    ex = jnp.exp(logits - m)
    aff = ex / jnp.sum(ex, axis=-1, keepdims=True)

    blk = aff.shape[0]
    lane = jax.lax.broadcasted_iota(jnp.int32, (blk, LANES), 1)
    gatemask = jnp.zeros((blk, LANES), jnp.bool_)
    work = aff
    for _ in range(TOPK):
        mx = jnp.max(work, axis=-1, keepdims=True)
        ismx = work == mx
        first = jnp.min(jnp.where(ismx, lane, jnp.int32(1 << 30)),
                        axis=-1, keepdims=True)
        chosen = lane == first
        gatemask = jnp.logical_or(gatemask, chosen)
        work = jnp.where(chosen, _NEG, work)
    gates_ref[...] = jnp.where(gatemask, aff, 0.0)


def _k2_body(xb_ref, w1_ref, b1_ref, w2_ref, b2_ref, gates_ref, y0_ref,
             out_ref, acc_ref):
    e = pl.program_id(1)

    @pl.when(e == 0)
    def _():
        acc_ref[...] = y0_ref[...]

    h = jnp.dot(xb_ref[...], w1_ref[0], preferred_element_type=jnp.float32)
    h = _gelu_exact(h + b1_ref[0])
    eo = jnp.dot(h.astype(jnp.bfloat16), w2_ref[0],
                 preferred_element_type=jnp.float32) + b2_ref[0]
    blk = eo.shape[0]
    lane = jax.lax.broadcasted_iota(jnp.int32, (blk, LANES), 1)
    g = jnp.sum(jnp.where(lane == e, gates_ref[...], 0.0),
                axis=1, keepdims=True)
    acc_ref[...] += g * eo

    @pl.when(e == NE - 1)
    def _():
        out_ref[...] = acc_ref[...]


def kernel(x, W1s, b1s, W2s, b2s, W1r, b1r, W2r, b2r, Wr, br):
    x2 = x.reshape(SEQ, DIM)
    xb = x2.astype(jnp.bfloat16)
    w1s = W1s.astype(jnp.bfloat16)
    w2s = W2s.astype(jnp.bfloat16)
    w1r = W1r.astype(jnp.bfloat16)
    w2r = W2r.astype(jnp.bfloat16)
    wr_p = jnp.pad(Wr, ((0, 0), (0, LANES - NE)))
    br_p = jnp.pad(br, (0, LANES - NE), constant_values=_NEG).reshape(1, LANES)
    b1s2 = b1s.reshape(1, INTER)
    b2s2 = b2s.reshape(1, DIM)

    B1 = 256
    y0, gates = pl.pallas_call(
        _k1_body,
        grid=(SEQ // B1,),
        in_specs=[
            pl.BlockSpec((B1, DIM), lambda i: (i, 0)),
            pl.BlockSpec((B1, DIM), lambda i: (i, 0)),
            pl.BlockSpec((DIM, INTER), lambda i: (0, 0)),
            pl.BlockSpec((1, INTER), lambda i: (0, 0)),
            pl.BlockSpec((INTER, DIM), lambda i: (0, 0)),
            pl.BlockSpec((1, DIM), lambda i: (0, 0)),
            pl.BlockSpec((DIM, LANES), lambda i: (0, 0)),
            pl.BlockSpec((1, LANES), lambda i: (0, 0)),
        ],
        out_specs=[
            pl.BlockSpec((B1, DIM), lambda i: (i, 0)),
            pl.BlockSpec((B1, LANES), lambda i: (i, 0)),
        ],
        out_shape=[
            jax.ShapeDtypeStruct((SEQ, DIM), jnp.float32),
            jax.ShapeDtypeStruct((SEQ, LANES), jnp.float32),
        ],
    )(x2, xb, w1s, b1s2, w2s, b2s2, wr_p, br_p)

    B2 = 1024
    out = pl.pallas_call(
        _k2_body,
        grid=(SEQ // B2, NE),
        in_specs=[
            pl.BlockSpec((B2, DIM), lambda i, e: (i, 0)),
            pl.BlockSpec((1, DIM, INTER), lambda i, e: (e, 0, 0)),
            pl.BlockSpec((1, 1, INTER), lambda i, e: (e, 0, 0)),
            pl.BlockSpec((1, INTER, DIM), lambda i, e: (e, 0, 0)),
            pl.BlockSpec((1, 1, DIM), lambda i, e: (e, 0, 0)),
            pl.BlockSpec((B2, LANES), lambda i, e: (i, 0)),
            pl.BlockSpec((B2, DIM), lambda i, e: (i, 0)),
        ],
        out_specs=pl.BlockSpec((B2, DIM), lambda i, e: (i, 0)),
        out_shape=jax.ShapeDtypeStruct((SEQ, DIM), jnp.float32),
        scratch_shapes=[pltpu.VMEM((B2, DIM), jnp.float32)],
    )(xb, w1r, b1r.reshape(NE, 1, INTER), w2r, b2r.reshape(NE, 1, DIM),
      gates, y0)

    return out.reshape(1, SEQ, DIM)
